# baseline (speedup 1.0000x reference)
import jax
import jax.numpy as jnp
from jax import lax
from jax.experimental import pallas as pl
from jax.experimental.pallas import tpu as pltpu
from jax.experimental.pallas import tpu_sc as plsc

BATCH = 16384
FIELDS = 26
FEATURES = 64

NC = 2
NS = 16
NW = NC * NS
BLK = 1024
NBLK = BATCH // BLK
NUNIT = FIELDS * NBLK
UPW = NUNIT // NW
SB = 128
NSB = BLK // SB
NBUF = 2


def _embed_body(idx_hbm, table_hbm, out_hbm, idx_v, tbr_v, tbt_v, *sems):
    gsems = sems[:NBUF]
    wsems = sems[NBUF:]
    wid = lax.axis_index("s") * NC + lax.axis_index("c")
    lanes = jnp.arange(16, dtype=jnp.int32)

    def issue(s, b):
        for q in range(SB // 16):
            iv = idx_v[pl.ds(s * SB + q * 16, 16)]
            for l in range(16):
                pltpu.async_copy(
                    table_hbm.at[pl.ds(iv[l], 1)],
                    tbr_v.at[b, pl.ds(q * 16 + l, 1)],
                    gsems[b],
                )

    def drain(b):
        pltpu.make_async_copy(
            table_hbm.at[pl.ds(0, SB)], tbr_v.at[b], gsems[b]
        ).wait()

    def transpose(b):
        src = tbr_v.at[b]
        dst = tbt_v.at[b]
        for c in range(FEATURES):
            col = jnp.full((16,), c, dtype=jnp.int32)
            for k0 in range(0, SB, 16):
                v16 = plsc.load_gather(src, [k0 + lanes, col])
                dst[c, pl.ds(k0, 16)] = v16

    def wait_flush(b):
        pltpu.make_async_copy(
            out_hbm.at[0, :, pl.ds(0, SB)],
            out_hbm.at[0, :, pl.ds(0, SB)],
            wsems[b],
        ).wait()

    def unit(uu, carry):
        u = uu * NW + wid
        f = u // NBLK
        blk = u % NBLK

        pltpu.sync_copy(idx_hbm.at[u], idx_v)

        issue(0, 0)

        def subblock(sg, c2):
            for bb in range(NBUF):
                s = sg * NBUF + bb
                nb = (bb + 1) % NBUF
                drain(bb)

                @pl.when(s + 1 < NSB)
                def _():
                    issue(s + 1, nb)

                @pl.when(s >= NBUF)
                def _():
                    wait_flush(bb)

                transpose(bb)
                pltpu.async_copy(
                    tbt_v.at[bb],
                    out_hbm.at[f, :, pl.ds(blk * BLK + s * SB, SB)],
                    wsems[bb],
                )

            return c2

        lax.fori_loop(0, NSB // NBUF, subblock, 0)

        for bb in range(NBUF):
            wait_flush(bb)
        return carry

    lax.fori_loop(0, UPW, unit, 0)


@jax.jit
def _run(idxt, table):
    f = pl.kernel(
        _embed_body,
        out_type=jax.ShapeDtypeStruct((FIELDS, FEATURES, BATCH), jnp.float32),
        mesh=plsc.VectorSubcoreMesh(core_axis_name="c", subcore_axis_name="s"),
        scratch_types=[
            pltpu.VMEM((BLK,), jnp.int32),
            pltpu.VMEM((NBUF, SB, FEATURES), jnp.float32),
            pltpu.VMEM((NBUF, FEATURES, SB), jnp.float32),
        ]
        + [pltpu.SemaphoreType.DMA] * (2 * NBUF),
        compiler_params=pltpu.CompilerParams(needs_layout_passes=False),
    )
    outp = f(idxt, table)
    return jnp.transpose(outp, (2, 0, 1))


def kernel(inputs, embedding):
    idxt = inputs.astype(jnp.int32).T.reshape(NUNIT, BLK)
    return _run(idxt, embedding)


# speedup vs baseline: 1.6137x; 1.6137x over previous
import jax
import jax.numpy as jnp
from jax import lax
from jax.experimental import pallas as pl
from jax.experimental.pallas import tpu as pltpu
from jax.experimental.pallas import tpu_sc as plsc

BATCH = 16384
FIELDS = 26
FEATURES = 64

NC = 2
NS = 16
NW = NC * NS
BPW_B = BATCH // NW
CB = 8
CH = CB * FIELDS
NCHUNK = BPW_B // CB
NBUF = 4


def _embed_body(idx_hbm, table_hbm, out_hbm, idx_v, rows_v, *sems):
    gsems = sems[:NBUF]
    wsems = sems[NBUF:]
    wid = lax.axis_index("s") * NC + lax.axis_index("c")
    b0 = wid * BPW_B

    pltpu.sync_copy(idx_hbm.at[wid], idx_v)

    def issue(c, b):
        for g in range(CH // 16):
            iv = idx_v[c, pl.ds(g * 16, 16)]
            for l in range(16):
                pltpu.async_copy(
                    table_hbm.at[pl.ds(iv[l], 1)],
                    rows_v.at[b, pl.ds(g * 16 + l, 1)],
                    gsems[b],
                )

    def drain(b):
        pltpu.make_async_copy(
            table_hbm.at[pl.ds(0, CH)], rows_v.at[b], gsems[b]
        ).wait()

    def wait_write(b):
        pltpu.make_async_copy(
            out_hbm.at[pl.ds(0, CB)], out_hbm.at[pl.ds(0, CB)], wsems[b]
        ).wait()

    issue(0, 0)

    def group(g, carry):
        for bb in range(NBUF):
            c = g * NBUF + bb
            nb = (bb + 1) % NBUF
            drain(bb)

            @pl.when(c + 1 < NCHUNK)
            def _():
                @pl.when(c >= NBUF - 1)
                def _():
                    wait_write(nb)

                issue(c + 1, nb)

            for k in range(CB):
                pltpu.async_copy(
                    rows_v.at[bb, pl.ds(k * FIELDS, FIELDS)],
                    out_hbm.at[b0 + c * CB + k],
                    wsems[bb],
                )
        return carry

    lax.fori_loop(0, NCHUNK // NBUF, group, 0)

    for bb in range(NBUF):
        wait_write(bb)


@jax.jit
def _run(idx, table):
    f = pl.kernel(
        _embed_body,
        out_type=jax.ShapeDtypeStruct((BATCH, FIELDS, FEATURES), jnp.float32),
        mesh=plsc.VectorSubcoreMesh(core_axis_name="c", subcore_axis_name="s"),
        scratch_types=[
            pltpu.VMEM((NCHUNK, CH), jnp.int32),
            pltpu.VMEM((NBUF, CH, FEATURES), jnp.float32),
        ]
        + [pltpu.SemaphoreType.DMA] * (2 * NBUF),
    )
    return f(idx, table)


def kernel(inputs, embedding):
    idx = inputs.astype(jnp.int32).reshape(NW, NCHUNK, CH)
    return _run(idx, embedding)
